# baseline (speedup 1.0000x reference)
import functools

import jax
import jax.numpy as jnp
from jax import lax
from jax.experimental import pallas as pl
from jax.experimental.pallas import tpu as pltpu
from jax.experimental.pallas import tpu_sc as plsc

N = 10000
E = 160000
F_IN = 128
DIM = 32
EA = 16
B = 64

NC = 2
NS = 16
NW = NC * NS
CHUNK = 128
NPAD = 10240
ROWS_PER_SUB = NPAD // NS
EPAD = 163840

NSH = 2
ESH = EPAD // NSH
TILE_SH = ESH // NW
NCHUNKS_SH = TILE_SH // CHUNK
CPS = 10
SUP_SH = NCHUNKS_SH // CPS
ROWS_SUP = CPS * CHUNK

TILE_FULL = EPAD // NW
NCHUNKS_FULL = TILE_FULL // CHUNK
SUP_FULL = NCHUNKS_FULL // CPS

TE = 1024


def _sc_mesh():
    return plsc.VectorSubcoreMesh(core_axis_name="c", subcore_axis_name="s",
                                  num_cores=NC, num_subcores=NS)


_SC_PARAMS = pltpu.CompilerParams(use_tc_tiling_on_sc=False)


def _sc_gather(table, src_idx3):
    @functools.partial(
        pl.kernel,
        out_type=jax.ShapeDtypeStruct((ESH, DIM), jnp.bfloat16),
        mesh=_sc_mesh(),
        scratch_types=[
            pltpu.VMEM((NCHUNKS_SH, CHUNK), jnp.int32),
            pltpu.VMEM((ROWS_SUP, DIM), jnp.bfloat16),
            pltpu.VMEM((ROWS_SUP, DIM), jnp.bfloat16),
            pltpu.SemaphoreType.DMA,
            pltpu.SemaphoreType.DMA,
        ],
        compiler_params=_SC_PARAMS,
    )
    def k(table_hbm, idx_hbm, out_hbm, idx_v, rows0, rows1, gsem, ssem):
        c = lax.axis_index("c")
        s = lax.axis_index("s")
        wid = s * NC + c
        base = wid * TILE_SH
        pltpu.sync_copy(idx_hbm.at[wid], idx_v)
        bufs = (rows0, rows1)
        store = [None, None]
        for sc_i in range(SUP_SH):
            buf = bufs[sc_i % 2]
            if store[sc_i % 2] is not None:
                store[sc_i % 2].wait()
            descs = [
                pltpu.async_copy(table_hbm.at[idx_v.at[sc_i * CPS + j]],
                                 buf.at[pl.ds(j * CHUNK, CHUNK)], gsem)
                for j in range(CPS)
            ]
            for d in descs:
                d.wait()
            store[sc_i % 2] = pltpu.async_copy(
                buf, out_hbm.at[pl.ds(base + sc_i * ROWS_SUP, ROWS_SUP)],
                ssem)
        for d in store:
            if d is not None:
                d.wait()

    return k(table, src_idx3)


def _sc_scatter_add(vals, dst_idx3, zero_rows):
    @functools.partial(
        pl.kernel,
        out_type=jax.ShapeDtypeStruct((NC, NPAD, DIM), jnp.float32),
        mesh=_sc_mesh(),
        scratch_types=[
            pltpu.VMEM((NCHUNKS_SH, CHUNK), jnp.int32),
            pltpu.VMEM((ROWS_SUP, DIM), jnp.float32),
            pltpu.VMEM((ROWS_SUP, DIM), jnp.float32),
            pltpu.VMEM_SHARED((NPAD, DIM), jnp.float32),
            pltpu.SemaphoreType.DMA,
            pltpu.SemaphoreType.DMA,
            pltpu.SemaphoreType.DMA,
        ],
        compiler_params=_SC_PARAMS,
    )
    def k(vals_hbm, idx_hbm, zero_hbm, out_hbm, idx_v, val0, val1, agg_sh,
          vsem, wsem, zsem):
        c = lax.axis_index("c")
        s = lax.axis_index("s")
        wid = s * NC + c
        r0 = s * ROWS_PER_SUB
        zdesc = pltpu.async_copy(zero_hbm.at[pl.ds(r0, ROWS_PER_SUB)],
                                 agg_sh.at[pl.ds(r0, ROWS_PER_SUB)], zsem)
        pltpu.sync_copy(idx_hbm.at[wid], idx_v)
        base = wid * TILE_SH
        bufs = (val0, val1)
        loads = [None, None]
        loads[0] = pltpu.async_copy(
            vals_hbm.at[pl.ds(base, ROWS_SUP)], val0, vsem)
        zdesc.wait()
        plsc.subcore_barrier()
        for sc_i in range(SUP_SH):
            if sc_i + 1 < SUP_SH:
                loads[(sc_i + 1) % 2] = pltpu.async_copy(
                    vals_hbm.at[pl.ds(base + (sc_i + 1) * ROWS_SUP, ROWS_SUP)],
                    bufs[(sc_i + 1) % 2], vsem)
            loads[sc_i % 2].wait()
            buf = bufs[sc_i % 2]
            descs = [
                pltpu.async_copy(buf.at[pl.ds(j * CHUNK, CHUNK)],
                                 agg_sh.at[idx_v.at[sc_i * CPS + j]], wsem,
                                 add=True)
                for j in range(CPS)
            ]
            for d in descs:
                d.wait()
        plsc.subcore_barrier()
        pltpu.sync_copy(agg_sh.at[pl.ds(r0, ROWS_PER_SUB)],
                        out_hbm.at[c, pl.ds(r0, ROWS_PER_SUB)])

    return k(vals, dst_idx3, zero_rows)


def _sc_degree(dst_idx3, zero_rows, ones_chunk):
    @functools.partial(
        pl.kernel,
        out_type=jax.ShapeDtypeStruct((NC, NPAD, DIM), jnp.float32),
        mesh=_sc_mesh(),
        scratch_types=[
            pltpu.VMEM((NCHUNKS_FULL, CHUNK), jnp.int32),
            pltpu.VMEM((CHUNK, DIM), jnp.float32),
            pltpu.VMEM_SHARED((NPAD, DIM), jnp.float32),
            pltpu.SemaphoreType.DMA,
        ],
        compiler_params=_SC_PARAMS,
    )
    def k(idx_hbm, zero_hbm, ones_hbm, out_hbm, idx_v, val_v, agg_sh, sem):
        c = lax.axis_index("c")
        s = lax.axis_index("s")
        wid = s * NC + c
        r0 = s * ROWS_PER_SUB
        pltpu.sync_copy(zero_hbm.at[pl.ds(r0, ROWS_PER_SUB)],
                        agg_sh.at[pl.ds(r0, ROWS_PER_SUB)])
        pltpu.sync_copy(ones_hbm, val_v)
        pltpu.sync_copy(idx_hbm.at[wid], idx_v)
        plsc.subcore_barrier()
        for sc_i in range(SUP_FULL):
            descs = [
                pltpu.async_copy(val_v, agg_sh.at[idx_v.at[sc_i * CPS + j]],
                                 sem, add=True)
                for j in range(CPS)
            ]
            for d in descs:
                d.wait()
        plsc.subcore_barrier()
        pltpu.sync_copy(agg_sh.at[pl.ds(r0, ROWS_PER_SUB)],
                        out_hbm.at[c, pl.ds(r0, ROWS_PER_SUB)])

    return k(dst_idx3, zero_rows, ones_chunk)


def _tc_node_encode(x, w0t, b0r):
    def body(x_ref, w_ref, b_ref, o_ref, obf_ref):
        o = jnp.maximum(
            jnp.dot(x_ref[...], w_ref[...],
                    preferred_element_type=jnp.float32) + b_ref[...], 0.0)
        o_ref[...] = o
        obf_ref[...] = o.astype(jnp.bfloat16)

    return pl.pallas_call(
        body,
        out_shape=(jax.ShapeDtypeStruct((N, DIM), jnp.float32),
                   jax.ShapeDtypeStruct((N, DIM), jnp.bfloat16)),
    )(x, w0t, b0r)


def _tc_edge_mlp(ea, w1t, b1r):
    TEH = 2048
    def body(ea_ref, w_ref, b_ref, o_ref):
        h = jnp.maximum(
            jnp.dot(ea_ref[...], w_ref[...],
                    preferred_element_type=jnp.float32) + b_ref[...], 0.0)
        o_ref[...] = h.astype(jnp.bfloat16)

    return pl.pallas_call(
        body,
        grid=(EPAD // TEH,),
        in_specs=[
            pl.BlockSpec((TEH, EA), lambda i: (i, 0)),
            pl.BlockSpec((EA, F_IN), lambda i: (0, 0)),
            pl.BlockSpec((1, F_IN), lambda i: (0, 0)),
        ],
        out_specs=pl.BlockSpec((TEH, F_IN), lambda i: (i, 0)),
        out_shape=jax.ShapeDtypeStruct((EPAD, F_IN), jnp.bfloat16),
    )(ea, w1t, b1r)


def _tc_msg(hid_bf, xj_bf, w2t_bf, b2r, r_bf):
    def body(hid_ref, xj_ref, w2_ref, b2_ref, r_ref, o_ref):
        w = jnp.dot(hid_ref[...], w2_ref[...],
                    preferred_element_type=jnp.float32) + b2_ref[...]
        xr = jnp.dot(xj_ref[...], r_ref[...],
                     preferred_element_type=jnp.float32)
        q = w[:, 0:128] * xr[:, 0:128]
        for g in range(1, 8):
            q = q + w[:, 128 * g:128 * (g + 1)] * xr[:, 128 * g:128 * (g + 1)]
        o_ref[...] = (q[:, 0:32] + q[:, 32:64] + q[:, 64:96] + q[:, 96:128])

    return pl.pallas_call(
        body,
        grid=(ESH // TE,),
        in_specs=[
            pl.BlockSpec((TE, F_IN), lambda i: (i, 0)),
            pl.BlockSpec((TE, DIM), lambda i: (i, 0)),
            pl.BlockSpec((F_IN, DIM * DIM), lambda i: (0, 0)),
            pl.BlockSpec((1, DIM * DIM), lambda i: (0, 0)),
            pl.BlockSpec((DIM, DIM * DIM), lambda i: (0, 0)),
        ],
        out_specs=pl.BlockSpec((TE, DIM), lambda i: (i, 0)),
        out_shape=jax.ShapeDtypeStruct((ESH, DIM), jnp.float32),
    )(hid_bf, xj_bf, w2t_bf, b2r, r_bf)


def _tc_gru(h, aggs, degs, bconv_r, wit, wht, bir, bhr):
    def body(h_ref, a0_ref, a1_ref, a2_ref, a3_ref, d0_ref, d1_ref, bc_ref,
             wi_ref, wh_ref, bi_ref, bh_ref, o_ref, obf_ref):
        deg = jnp.maximum(d0_ref[...] + d1_ref[...], 1.0)
        asum = (a0_ref[...] + a1_ref[...]) + (a2_ref[...] + a3_ref[...])
        m = jnp.maximum(asum / deg + bc_ref[...], 0.0)
        hv = h_ref[...]
        gi = jnp.dot(m, wi_ref[...],
                     preferred_element_type=jnp.float32) + bi_ref[...]
        gh = jnp.dot(hv, wh_ref[...],
                     preferred_element_type=jnp.float32) + bh_ref[...]
        r = jax.nn.sigmoid(gi[:, 0:DIM] + gh[:, 0:DIM])
        z = jax.nn.sigmoid(gi[:, DIM:2 * DIM] + gh[:, DIM:2 * DIM])
        n = jnp.tanh(gi[:, 2 * DIM:] + r * gh[:, 2 * DIM:])
        o = (1.0 - z) * n + z * hv
        o_ref[...] = o
        obf_ref[...] = o.astype(jnp.bfloat16)

    return pl.pallas_call(
        body,
        out_shape=(jax.ShapeDtypeStruct((N, DIM), jnp.float32),
                   jax.ShapeDtypeStruct((N, DIM), jnp.bfloat16)),
    )(h, aggs[0], aggs[1], aggs[2], aggs[3], degs[0], degs[1], bconv_r,
      wit, wht, bir, bhr)


def _tc_set2set(outv, batch_col, wilt, whlt, bilr, bhlr):
    def body(out_ref, b_ref, wil_ref, whl_ref, bil_ref, bhl_ref, q_ref):
        ov = out_ref[...]
        seg = b_ref[...] == lax.broadcasted_iota(jnp.int32, (1, B), 1)
        segf = seg.astype(jnp.float32)
        q_star = jnp.zeros((B, 2 * DIM), jnp.float32)
        hl = jnp.zeros((B, DIM), jnp.float32)
        cl = jnp.zeros((B, DIM), jnp.float32)
        for _ in range(3):
            gates = (jnp.dot(q_star, wil_ref[...],
                             preferred_element_type=jnp.float32) + bil_ref[...]
                     + jnp.dot(hl, whl_ref[...],
                               preferred_element_type=jnp.float32)
                     + bhl_ref[...])
            ig = jax.nn.sigmoid(gates[:, 0:DIM])
            fg = jax.nn.sigmoid(gates[:, DIM:2 * DIM])
            gg = jnp.tanh(gates[:, 2 * DIM:3 * DIM])
            og = jax.nn.sigmoid(gates[:, 3 * DIM:])
            cl = fg * cl + ig * gg
            hl = og * jnp.tanh(cl)
            qn = jnp.dot(segf, hl, preferred_element_type=jnp.float32)
            e = jnp.sum(ov * qn, axis=1, keepdims=True)
            ee = jnp.where(seg, e, -jnp.inf)
            emax = jnp.max(ee, axis=0, keepdims=True)
            emax = jnp.where(jnp.isfinite(emax), emax, 0.0)
            emaxg = jnp.sum(jnp.where(seg, emax, 0.0), axis=1, keepdims=True)
            ex = jnp.exp(e - emaxg)
            esum = jnp.sum(segf * ex, axis=0, keepdims=True)
            esumg = jnp.sum(jnp.where(seg, esum, 0.0), axis=1, keepdims=True)
            a = ex / jnp.maximum(esumg, 1e-16)
            r_read = lax.dot_general(segf * a, ov,
                                     (((0,), (0,)), ((), ())),
                                     preferred_element_type=jnp.float32)
            q_star = jnp.concatenate([hl, r_read], axis=1)
        q_ref[...] = q_star

    return pl.pallas_call(
        body,
        out_shape=jax.ShapeDtypeStruct((B, 2 * DIM), jnp.float32),
    )(outv, batch_col, wilt, whlt, bilr, bhlr)


def kernel(x, edge_index, edge_attr, batch, W0, b0, We1, be1, We2, be2, bconv,
           Wih_g, Whh_g, bih_g, bhh_g, Wih_l, Whh_l, bih_l, bhh_l):
    f32 = jnp.float32
    src = edge_index[0]
    dst = edge_index[1]
    pad = EPAD - E
    src_p = jnp.concatenate([src, jnp.zeros((pad,), jnp.int32)])
    dst_p = jnp.concatenate([dst, jnp.full((pad,), N, jnp.int32)])
    src_sh = [src_p[i * ESH:(i + 1) * ESH].reshape(NW, NCHUNKS_SH, CHUNK)
              for i in range(NSH)]
    dst_sh = [dst_p[i * ESH:(i + 1) * ESH].reshape(NW, NCHUNKS_SH, CHUNK)
              for i in range(NSH)]
    dst_full = dst_p.reshape(NW, NCHUNKS_FULL, CHUNK)
    ea_p = jnp.concatenate([edge_attr, jnp.zeros((pad, EA), f32)], axis=0)
    zero_rows = jnp.zeros((NPAD, DIM), f32)
    ones_chunk = jnp.ones((CHUNK, DIM), f32)

    w0t = W0.T
    w1t = We1.T
    w2t_bf = We2.T.astype(jnp.bfloat16)
    r_bf = jnp.kron(jnp.eye(DIM, dtype=f32),
                    jnp.ones((1, DIM), f32)).astype(jnp.bfloat16)
    wit = Wih_g.T
    wht = Whh_g.T
    wilt = Wih_l.T
    whlt = Whh_l.T

    h, h_bf = _tc_node_encode(x, w0t, b0[None])
    hid_bf = _tc_edge_mlp(ea_p, w1t, be1[None])
    hid_sh = [hid_bf[i * ESH:(i + 1) * ESH] for i in range(NSH)]

    degp = None
    for rnd in range(3):
        xjs = [_sc_gather(h_bf, src_sh[i]) for i in range(NSH)]
        if rnd == 0:
            dep = (xjs[0][0, 0] * 0).astype(f32)
            degp = _sc_degree(dst_full, zero_rows + dep, ones_chunk)
        msgs = [_tc_msg(hid_sh[i], xjs[i], w2t_bf, be2[None], r_bf)
                for i in range(NSH)]
        aggp = [_sc_scatter_add(msgs[i], dst_sh[i], zero_rows)
                for i in range(NSH)]
        aggs = [aggp[0][0, :N], aggp[0][1, :N], aggp[1][0, :N], aggp[1][1, :N]]
        if rnd == 0:
            degs = (degp[0, :N], degp[1, :N])
        h, h_bf = _tc_gru(h, aggs, degs, bconv[None], wit, wht,
                          bih_g[None], bhh_g[None])

    q_star = _tc_set2set(h, batch[:, None], wilt, whlt,
                         bih_l[None], bhh_l[None])
    return q_star, h


# speedup vs baseline: 1.0309x; 1.0309x over previous
import functools

import jax
import jax.numpy as jnp
from jax import lax
from jax.experimental import pallas as pl
from jax.experimental.pallas import tpu as pltpu
from jax.experimental.pallas import tpu_sc as plsc

N = 10000
E = 160000
F_IN = 128
DIM = 32
EA = 16
B = 64

NC = 2
NS = 16
NW = NC * NS
CHUNK = 128
NPAD = 10240
ROWS_PER_SUB = NPAD // NS
EPAD = 163840

NSH = 2
ESH = EPAD // NSH
TILE_SH = ESH // NW
NCHUNKS_SH = TILE_SH // CHUNK
CPS = 10
SUP_SH = NCHUNKS_SH // CPS
ROWS_SUP = CPS * CHUNK

TILE_FULL = EPAD // NW
NCHUNKS_FULL = TILE_FULL // CHUNK
SUP_FULL = NCHUNKS_FULL // CPS

TE = 1024


def _sc_mesh():
    return plsc.VectorSubcoreMesh(core_axis_name="c", subcore_axis_name="s",
                                  num_cores=NC, num_subcores=NS)


_SC_PARAMS = pltpu.CompilerParams(use_tc_tiling_on_sc=False)


def _sc_gather(table, src_idx3):
    @functools.partial(
        pl.kernel,
        out_type=jax.ShapeDtypeStruct((ESH, DIM), jnp.bfloat16),
        mesh=_sc_mesh(),
        scratch_types=[
            pltpu.VMEM((NCHUNKS_SH, CHUNK), jnp.int32),
            pltpu.VMEM((ROWS_SUP, DIM), jnp.bfloat16),
            pltpu.VMEM((ROWS_SUP, DIM), jnp.bfloat16),
            pltpu.SemaphoreType.DMA,
            pltpu.SemaphoreType.DMA,
        ],
        compiler_params=_SC_PARAMS,
    )
    def k(table_hbm, idx_hbm, out_hbm, idx_v, rows0, rows1, gsem, ssem):
        c = lax.axis_index("c")
        s = lax.axis_index("s")
        wid = s * NC + c
        base = wid * TILE_SH
        pltpu.sync_copy(idx_hbm.at[wid], idx_v)
        bufs = (rows0, rows1)
        store = [None, None]
        for sc_i in range(SUP_SH):
            buf = bufs[sc_i % 2]
            if store[sc_i % 2] is not None:
                store[sc_i % 2].wait()
            descs = [
                pltpu.async_copy(table_hbm.at[idx_v.at[sc_i * CPS + j]],
                                 buf.at[pl.ds(j * CHUNK, CHUNK)], gsem)
                for j in range(CPS)
            ]
            for d in descs:
                d.wait()
            store[sc_i % 2] = pltpu.async_copy(
                buf, out_hbm.at[pl.ds(base + sc_i * ROWS_SUP, ROWS_SUP)],
                ssem)
        for d in store:
            if d is not None:
                d.wait()

    return k(table, src_idx3)


def _sc_scatter_add(vals, dst_idx3, zero_rows):
    @functools.partial(
        pl.kernel,
        out_type=jax.ShapeDtypeStruct((NC, NPAD, DIM), jnp.float32),
        mesh=_sc_mesh(),
        scratch_types=[
            pltpu.VMEM((NCHUNKS_SH, CHUNK), jnp.int32),
            pltpu.VMEM((ROWS_SUP, DIM), jnp.float32),
            pltpu.VMEM((ROWS_SUP, DIM), jnp.float32),
            pltpu.VMEM_SHARED((NPAD, DIM), jnp.float32),
            pltpu.SemaphoreType.DMA,
            pltpu.SemaphoreType.DMA,
            pltpu.SemaphoreType.DMA,
        ],
        compiler_params=_SC_PARAMS,
    )
    def k(vals_hbm, idx_hbm, zero_hbm, out_hbm, idx_v, val0, val1, agg_sh,
          vsem, wsem, zsem):
        c = lax.axis_index("c")
        s = lax.axis_index("s")
        wid = s * NC + c
        r0 = s * ROWS_PER_SUB
        zdesc = pltpu.async_copy(zero_hbm.at[pl.ds(r0, ROWS_PER_SUB)],
                                 agg_sh.at[pl.ds(r0, ROWS_PER_SUB)], zsem)
        pltpu.sync_copy(idx_hbm.at[wid], idx_v)
        base = wid * TILE_SH
        bufs = (val0, val1)
        loads = [None, None]
        loads[0] = pltpu.async_copy(
            vals_hbm.at[pl.ds(base, ROWS_SUP)], val0, vsem)
        zdesc.wait()
        plsc.subcore_barrier()
        for sc_i in range(SUP_SH):
            if sc_i + 1 < SUP_SH:
                loads[(sc_i + 1) % 2] = pltpu.async_copy(
                    vals_hbm.at[pl.ds(base + (sc_i + 1) * ROWS_SUP, ROWS_SUP)],
                    bufs[(sc_i + 1) % 2], vsem)
            loads[sc_i % 2].wait()
            buf = bufs[sc_i % 2]
            descs = [
                pltpu.async_copy(buf.at[pl.ds(j * CHUNK, CHUNK)],
                                 agg_sh.at[idx_v.at[sc_i * CPS + j]], wsem,
                                 add=True)
                for j in range(CPS)
            ]
            for d in descs:
                d.wait()
        plsc.subcore_barrier()
        pltpu.sync_copy(agg_sh.at[pl.ds(r0, ROWS_PER_SUB)],
                        out_hbm.at[c, pl.ds(r0, ROWS_PER_SUB)])

    return k(vals, dst_idx3, zero_rows)


def _sc_degree(dst_idx3, zero_rows, ones_chunk):
    @functools.partial(
        pl.kernel,
        out_type=jax.ShapeDtypeStruct((NC, NPAD, DIM), jnp.float32),
        mesh=_sc_mesh(),
        scratch_types=[
            pltpu.VMEM((NCHUNKS_FULL, CHUNK), jnp.int32),
            pltpu.VMEM((CHUNK, DIM), jnp.float32),
            pltpu.VMEM_SHARED((NPAD, DIM), jnp.float32),
            pltpu.SemaphoreType.DMA,
        ],
        compiler_params=_SC_PARAMS,
    )
    def k(idx_hbm, zero_hbm, ones_hbm, out_hbm, idx_v, val_v, agg_sh, sem):
        c = lax.axis_index("c")
        s = lax.axis_index("s")
        wid = s * NC + c
        r0 = s * ROWS_PER_SUB
        pltpu.sync_copy(zero_hbm.at[pl.ds(r0, ROWS_PER_SUB)],
                        agg_sh.at[pl.ds(r0, ROWS_PER_SUB)])
        pltpu.sync_copy(ones_hbm, val_v)
        pltpu.sync_copy(idx_hbm.at[wid], idx_v)
        plsc.subcore_barrier()
        for sc_i in range(SUP_FULL):
            descs = [
                pltpu.async_copy(val_v, agg_sh.at[idx_v.at[sc_i * CPS + j]],
                                 sem, add=True)
                for j in range(CPS)
            ]
            for d in descs:
                d.wait()
        plsc.subcore_barrier()
        pltpu.sync_copy(agg_sh.at[pl.ds(r0, ROWS_PER_SUB)],
                        out_hbm.at[c, pl.ds(r0, ROWS_PER_SUB)])

    return k(dst_idx3, zero_rows, ones_chunk)


def _tc_node_encode(x, w0t, b0r):
    def body(x_ref, w_ref, b_ref, o_ref, obf_ref):
        o = jnp.maximum(
            jnp.dot(x_ref[...], w_ref[...],
                    preferred_element_type=jnp.float32) + b_ref[...], 0.0)
        o_ref[...] = o
        obf_ref[...] = o.astype(jnp.bfloat16)

    return pl.pallas_call(
        body,
        out_shape=(jax.ShapeDtypeStruct((N, DIM), jnp.float32),
                   jax.ShapeDtypeStruct((N, DIM), jnp.bfloat16)),
    )(x, w0t, b0r)


def _tc_edge_mlp(ea, w1t, b1r):
    TEH = 2048
    def body(ea_ref, w_ref, b_ref, o_ref):
        h = jnp.maximum(
            jnp.dot(ea_ref[...], w_ref[...],
                    preferred_element_type=jnp.float32) + b_ref[...], 0.0)
        o_ref[...] = h.astype(jnp.bfloat16)

    return pl.pallas_call(
        body,
        grid=(EPAD // TEH,),
        in_specs=[
            pl.BlockSpec((TEH, EA), lambda i: (i, 0)),
            pl.BlockSpec((EA, F_IN), lambda i: (0, 0)),
            pl.BlockSpec((1, F_IN), lambda i: (0, 0)),
        ],
        out_specs=pl.BlockSpec((TEH, F_IN), lambda i: (i, 0)),
        out_shape=jax.ShapeDtypeStruct((EPAD, F_IN), jnp.bfloat16),
    )(ea, w1t, b1r)


def _tc_msg(hid_bf, xj_bf, w2t_bf, b2r, r_bf, shard):
    def body(hid_ref, xj_ref, w2_ref, b2_ref, r_ref, o_ref):
        w = jnp.dot(hid_ref[...], w2_ref[...],
                    preferred_element_type=jnp.float32) + b2_ref[...]
        xr = jnp.dot(xj_ref[...], r_ref[...],
                     preferred_element_type=jnp.float32)
        q = w[:, 0:128] * xr[:, 0:128]
        for g in range(1, 8):
            q = q + w[:, 128 * g:128 * (g + 1)] * xr[:, 128 * g:128 * (g + 1)]
        o_ref[...] = (q[:, 0:32] + q[:, 32:64] + q[:, 64:96] + q[:, 96:128])

    off = shard * (ESH // TE)
    return pl.pallas_call(
        body,
        grid=(ESH // TE,),
        in_specs=[
            pl.BlockSpec((TE, F_IN), lambda i: (i + off, 0)),
            pl.BlockSpec((TE, DIM), lambda i: (i, 0)),
            pl.BlockSpec((F_IN, DIM * DIM), lambda i: (0, 0)),
            pl.BlockSpec((1, DIM * DIM), lambda i: (0, 0)),
            pl.BlockSpec((DIM, DIM * DIM), lambda i: (0, 0)),
        ],
        out_specs=pl.BlockSpec((TE, DIM), lambda i: (i, 0)),
        out_shape=jax.ShapeDtypeStruct((ESH, DIM), jnp.float32),
    )(hid_bf, xj_bf, w2t_bf, b2r, r_bf)


def _tc_gru(h, agg_a, agg_b, degp, bconv_r, wit, wht, bir, bhr):
    def body(h_ref, a_ref, b_ref, d_ref, bc_ref,
             wi_ref, wh_ref, bi_ref, bh_ref, o_ref, obf_ref):
        deg = jnp.maximum(d_ref[0, :N] + d_ref[1, :N], 1.0)
        asum = ((a_ref[0, :N] + a_ref[1, :N])
                + (b_ref[0, :N] + b_ref[1, :N]))
        m = jnp.maximum(asum / deg + bc_ref[...], 0.0)
        hv = h_ref[...]
        gi = jnp.dot(m, wi_ref[...],
                     preferred_element_type=jnp.float32) + bi_ref[...]
        gh = jnp.dot(hv, wh_ref[...],
                     preferred_element_type=jnp.float32) + bh_ref[...]
        r = jax.nn.sigmoid(gi[:, 0:DIM] + gh[:, 0:DIM])
        z = jax.nn.sigmoid(gi[:, DIM:2 * DIM] + gh[:, DIM:2 * DIM])
        n = jnp.tanh(gi[:, 2 * DIM:] + r * gh[:, 2 * DIM:])
        o = (1.0 - z) * n + z * hv
        o_ref[...] = o
        obf_ref[...] = o.astype(jnp.bfloat16)

    return pl.pallas_call(
        body,
        out_shape=(jax.ShapeDtypeStruct((N, DIM), jnp.float32),
                   jax.ShapeDtypeStruct((N, DIM), jnp.bfloat16)),
    )(h, agg_a, agg_b, degp, bconv_r, wit, wht, bir, bhr)


def _tc_set2set(outv, batch_col, wilt, whlt, bilr, bhlr):
    def body(out_ref, b_ref, wil_ref, whl_ref, bil_ref, bhl_ref, q_ref):
        ov = out_ref[...]
        seg = b_ref[...] == lax.broadcasted_iota(jnp.int32, (1, B), 1)
        segf = seg.astype(jnp.float32)
        q_star = jnp.zeros((B, 2 * DIM), jnp.float32)
        hl = jnp.zeros((B, DIM), jnp.float32)
        cl = jnp.zeros((B, DIM), jnp.float32)
        for _ in range(3):
            gates = (jnp.dot(q_star, wil_ref[...],
                             preferred_element_type=jnp.float32) + bil_ref[...]
                     + jnp.dot(hl, whl_ref[...],
                               preferred_element_type=jnp.float32)
                     + bhl_ref[...])
            ig = jax.nn.sigmoid(gates[:, 0:DIM])
            fg = jax.nn.sigmoid(gates[:, DIM:2 * DIM])
            gg = jnp.tanh(gates[:, 2 * DIM:3 * DIM])
            og = jax.nn.sigmoid(gates[:, 3 * DIM:])
            cl = fg * cl + ig * gg
            hl = og * jnp.tanh(cl)
            qn = jnp.dot(segf, hl, preferred_element_type=jnp.float32)
            e = jnp.sum(ov * qn, axis=1, keepdims=True)
            ee = jnp.where(seg, e, -jnp.inf)
            emax = jnp.max(ee, axis=0, keepdims=True)
            emax = jnp.where(jnp.isfinite(emax), emax, 0.0)
            emaxg = jnp.sum(jnp.where(seg, emax, 0.0), axis=1, keepdims=True)
            ex = jnp.exp(e - emaxg)
            esum = jnp.sum(segf * ex, axis=0, keepdims=True)
            esumg = jnp.sum(jnp.where(seg, esum, 0.0), axis=1, keepdims=True)
            a = ex / jnp.maximum(esumg, 1e-16)
            r_read = lax.dot_general(segf * a, ov,
                                     (((0,), (0,)), ((), ())),
                                     preferred_element_type=jnp.float32)
            q_star = jnp.concatenate([hl, r_read], axis=1)
        q_ref[...] = q_star

    return pl.pallas_call(
        body,
        out_shape=jax.ShapeDtypeStruct((B, 2 * DIM), jnp.float32),
    )(outv, batch_col, wilt, whlt, bilr, bhlr)


def kernel(x, edge_index, edge_attr, batch, W0, b0, We1, be1, We2, be2, bconv,
           Wih_g, Whh_g, bih_g, bhh_g, Wih_l, Whh_l, bih_l, bhh_l):
    f32 = jnp.float32
    src = edge_index[0]
    dst = edge_index[1]
    pad = EPAD - E
    src_p = jnp.concatenate([src, jnp.zeros((pad,), jnp.int32)])
    dst_p = jnp.concatenate([dst, jnp.full((pad,), N, jnp.int32)])
    src_sh = [src_p[i * ESH:(i + 1) * ESH].reshape(NW, NCHUNKS_SH, CHUNK)
              for i in range(NSH)]
    dst_sh = [dst_p[i * ESH:(i + 1) * ESH].reshape(NW, NCHUNKS_SH, CHUNK)
              for i in range(NSH)]
    dst_full = dst_p.reshape(NW, NCHUNKS_FULL, CHUNK)
    ea_p = jnp.concatenate([edge_attr.astype(jnp.bfloat16),
                            jnp.zeros((pad, EA), jnp.bfloat16)], axis=0)
    zero_rows = jnp.zeros((NPAD, DIM), f32)
    ones_chunk = jnp.ones((CHUNK, DIM), f32)

    w0t = W0.T
    w1t = We1.T.astype(jnp.bfloat16)
    w2t_bf = We2.T.astype(jnp.bfloat16)
    r_bf = jnp.kron(jnp.eye(DIM, dtype=f32),
                    jnp.ones((1, DIM), f32)).astype(jnp.bfloat16)
    wit = Wih_g.T
    wht = Whh_g.T
    wilt = Wih_l.T
    whlt = Whh_l.T

    h, h_bf = _tc_node_encode(x, w0t, b0[None])
    hid_bf = _tc_edge_mlp(ea_p, w1t, be1[None])

    degp = None
    for rnd in range(3):
        xjs = [_sc_gather(h_bf, src_sh[i]) for i in range(NSH)]
        if rnd == 0:
            dep = (xjs[0][0, 0] * 0).astype(f32)
            degp = _sc_degree(dst_full, zero_rows + dep, ones_chunk)
        msgs = [_tc_msg(hid_bf, xjs[i], w2t_bf, be2[None], r_bf, i)
                for i in range(NSH)]
        aggp = [_sc_scatter_add(msgs[i], dst_sh[i], zero_rows)
                for i in range(NSH)]
        h, h_bf = _tc_gru(h, aggp[0], aggp[1], degp, bconv[None], wit, wht,
                          bih_g[None], bhh_g[None])

    q_star = _tc_set2set(h, batch[:, None], wilt, whlt,
                         bih_l[None], bhh_l[None])
    return q_star, h
